# Initial kernel scaffold; baseline (speedup 1.0000x reference)
import jax
import jax.numpy as jnp
from jax.experimental import pallas as pl


def kernel(feat, graph_ids):
    raise NotImplementedError("write your pallas kernel here")


# speedup vs baseline: 16.3393x; 16.3393x over previous
import functools

import jax
import jax.numpy as jnp
from jax.experimental import pallas as pl
from jax.experimental.pallas import tpu as pltpu
from jax.experimental.pallas import tpu_sc as plsc

N = 100000
D = 128
B = 100
K = 64

BLK = 1024
GRID = (N + BLK - 1) // BLK
NPAD = GRID * BLK
SROWS = NPAD // 128
WIN = 16
GB = 128
NEG = float("-inf")

NW = 32
RPW = (B * K) // NW


def _rowmax_body(feat_ref, gid_ref, scores_ref, ends_ref, acc_ref):
    i = pl.program_id(0)
    x = feat_ref[...]
    m = jnp.max(x.reshape(BLK // 128, 128, 128), axis=-1)
    scores_ref[...] = m[None]

    g = gid_ref[...][0]
    bv = jax.lax.broadcasted_iota(jnp.int32, (GB, 1, 1), 0)
    cnt = jnp.sum((g[None] <= bv).astype(jnp.int32), axis=(1, 2))

    @pl.when(i == 0)
    def _():
        acc_ref[...] = jnp.zeros((8, GB), jnp.int32)

    acc_ref[0:1, :] = acc_ref[0:1, :] + cnt.reshape(1, GB)

    @pl.when(i == GRID - 1)
    def _():
        ends_ref[...] = jnp.broadcast_to(acc_ref[0:1, :], (8, GB))


def _rowmax(feat, gpad):
    return pl.pallas_call(
        _rowmax_body,
        grid=(GRID,),
        in_specs=[
            pl.BlockSpec((BLK, 128), lambda i: (i, 0)),
            pl.BlockSpec((1, 8, 128), lambda i: (i, 0, 0)),
        ],
        out_specs=[
            pl.BlockSpec((1, 8, 128), lambda i: (i, 0, 0)),
            pl.BlockSpec((8, GB), lambda i: (0, 0)),
        ],
        out_shape=[
            jax.ShapeDtypeStruct((GRID, 8, 128), jnp.float32),
            jax.ShapeDtypeStruct((8, GB), jnp.int32),
        ],
        scratch_shapes=[pltpu.VMEM((8, GB), jnp.int32)],
    )(feat, gpad)


def _topk_body(scores_ref, ends_s_ref, ends_v_ref, out_ref, vals_ref):
    ev = ends_v_ref[0:1, :]
    sv = jnp.concatenate([jnp.zeros((1, 1), jnp.int32), ev[:, : GB - 1]], axis=1)
    r0v = jnp.clip(sv // 128, 0, SROWS - WIN)

    vals_ref[...] = jnp.full((GB, WIN, 128), NEG, jnp.float32)

    def build(b, carry):
        e = ends_s_ref[0, b]
        s = jnp.where(b == 0, 0, ends_s_ref[0, jnp.maximum(b - 1, 0)])
        r0 = jnp.clip(s // 128, 0, SROWS - WIN)
        win = scores_ref[pl.ds(r0, WIN), :]
        pos = (r0 * 128
               + jax.lax.broadcasted_iota(jnp.int32, (WIN, 128), 0) * 128
               + jax.lax.broadcasted_iota(jnp.int32, (WIN, 128), 1))
        mv = jnp.where((pos >= s) & (pos < e), win, NEG)
        vals_ref[pl.ds(b, 1)] = mv[None]
        return carry

    jax.lax.fori_loop(0, B, build, 0)
    vt = jnp.swapaxes(vals_ref[...].reshape(GB, WIN * 128), 0, 1)

    posi = jax.lax.broadcasted_iota(jnp.int32, (WIN * 128, 1), 0)

    def step(k, carry):
        m = jnp.max(carry, axis=0, keepdims=True)
        cand = jnp.where(carry == m, posi, WIN * 128)
        wp = jnp.min(cand, axis=0, keepdims=True)
        out_ref[pl.ds(k, 1), :] = r0v * 128 + wp
        return jnp.where(posi == wp, NEG, carry)

    jax.lax.fori_loop(0, K, step, vt)


def _topk(scores2, ends):
    return pl.pallas_call(
        _topk_body,
        grid=(1,),
        in_specs=[
            pl.BlockSpec((SROWS, 128), lambda i: (0, 0)),
            pl.BlockSpec(memory_space=pltpu.SMEM),
            pl.BlockSpec((8, GB), lambda i: (0, 0)),
        ],
        out_specs=pl.BlockSpec((K, GB), lambda i: (0, 0)),
        out_shape=jax.ShapeDtypeStruct((K, GB), jnp.int32),
        scratch_shapes=[pltpu.VMEM((GB, WIN, 128), jnp.float32)],
    )(scores2, ends, ends)


def _sc_gather(feat, idx):
    mesh = plsc.VectorSubcoreMesh(core_axis_name="c", subcore_axis_name="s")

    @functools.partial(
        pl.kernel,
        mesh=mesh,
        out_type=jax.ShapeDtypeStruct((B * K, D), jnp.float32),
        scratch_types=[
            pltpu.VMEM((RPW,), jnp.int32),
            pltpu.VMEM((RPW, D), jnp.float32),
            pltpu.SemaphoreType.DMA,
        ],
    )
    def k(feat_hbm, idx_hbm, out_hbm, idx_v, rows_v, sem):
        wid = jax.lax.axis_index("s") * 2 + jax.lax.axis_index("c")
        base = wid * RPW
        pltpu.sync_copy(idx_hbm.at[pl.ds(base, RPW)], idx_v)
        pltpu.async_copy(feat_hbm.at[idx_v], rows_v, sem).wait()
        pltpu.sync_copy(rows_v, out_hbm.at[pl.ds(base, RPW)])

    return k(feat, idx)


def _sort_body(x_ref, o_ref):
    x = x_ref[...]
    lane = jax.lax.broadcasted_iota(jnp.int32, (1, 128), 1)
    for kk in (2, 4, 8, 16, 32, 64, 128):
        j = kk // 2
        while j >= 1:
            bitj = (lane & j) != 0
            up = (lane & kk) == 0
            rolled_m = jnp.concatenate([x[:, j:], x[:, :j]], axis=1)
            rolled_p = jnp.concatenate([x[:, 128 - j:], x[:, : 128 - j]], axis=1)
            partner = jnp.where(bitj, rolled_p, rolled_m)
            take_min = jnp.logical_xor(bitj, up)
            x = jnp.where(take_min, jnp.minimum(x, partner),
                          jnp.maximum(x, partner))
            j //= 2
    o_ref[...] = x


def _sort_rows(rows):
    blk = 800
    return pl.pallas_call(
        _sort_body,
        grid=((B * K) // blk,),
        in_specs=[pl.BlockSpec((blk, 128), lambda i: (i, 0))],
        out_specs=pl.BlockSpec((blk, 128), lambda i: (i, 0)),
        out_shape=jax.ShapeDtypeStruct((B * K, 128), jnp.float32),
    )(rows)


def kernel(feat, graph_ids):
    gids = graph_ids.astype(jnp.int32)
    gpad = jnp.pad(gids, (0, NPAD - N), constant_values=B).reshape(GRID, 8, 128)
    scores3, ends = _rowmax(feat, gpad)
    scores2 = scores3.reshape(SROWS, 128)
    picks = _topk(scores2, ends)
    idx = picks[:, :B].T.reshape(B * K)
    rows = _sc_gather(feat, idx)
    srt = _sort_rows(rows)
    return srt.reshape(B, K * D)


# speedup vs baseline: 17.2516x; 1.0558x over previous
import functools

import jax
import jax.numpy as jnp
from jax.experimental import pallas as pl
from jax.experimental.pallas import tpu as pltpu
from jax.experimental.pallas import tpu_sc as plsc

N = 100000
D = 128
B = 100
K = 64

BLK = 1024
GRID = (N + BLK - 1) // BLK
NPAD = GRID * BLK
SROWS = NPAD // 128
WIN = 16
GB = 128
NEG = float("-inf")

NW = 32
RPW = (B * K) // NW


def _rowmax_body(feat_ref, gid_ref, scores_ref, ends_ref, acc_ref):
    i = pl.program_id(0)
    x = feat_ref[...]
    scores_ref[...] = jnp.max(x.reshape(BLK // 128, 128, 128), axis=-1)

    g = gid_ref[...][0]
    bv = jax.lax.broadcasted_iota(jnp.int32, (GB, 1, 1), 0)
    inc = (g[None] <= bv).astype(jnp.int32)

    @pl.when(i == 0)
    def _():
        acc_ref[...] = inc

    @pl.when(i > 0)
    def _():
        acc_ref[...] = acc_ref[...] + inc

    @pl.when(i == GRID - 1)
    def _():
        cnt = jnp.sum(acc_ref[...], axis=(1, 2)).reshape(1, GB)
        ends_ref[...] = jnp.broadcast_to(cnt, (8, GB))


def _rowmax(feat, gpad):
    return pl.pallas_call(
        _rowmax_body,
        grid=(GRID,),
        in_specs=[
            pl.BlockSpec((BLK, 128), lambda i: (i, 0)),
            pl.BlockSpec((1, 8, 128), lambda i: (i, 0, 0)),
        ],
        out_specs=[
            pl.BlockSpec((8, 128), lambda i: (i, 0)),
            pl.BlockSpec((8, GB), lambda i: (0, 0)),
        ],
        out_shape=[
            jax.ShapeDtypeStruct((SROWS, 128), jnp.float32),
            jax.ShapeDtypeStruct((8, GB), jnp.int32),
        ],
        scratch_shapes=[pltpu.VMEM((GB, 8, 128), jnp.int32)],
    )(feat, gpad)


def _topk_body(scores_ref, ends_s_ref, ends_v_ref, out_ref, vals_ref, picks_ref):
    ev = ends_v_ref[0:1, :]
    sv = jnp.concatenate([jnp.zeros((1, 1), jnp.int32), ev[:, : GB - 1]], axis=1)
    r0v = jnp.clip(sv // 128, 0, SROWS - WIN)

    vals_ref[...] = jnp.full((GB, WIN, 128), NEG, jnp.float32)

    def build(b, carry):
        e = ends_s_ref[0, b]
        s = jnp.where(b == 0, 0, ends_s_ref[0, jnp.maximum(b - 1, 0)])
        r0 = jnp.clip(s // 128, 0, SROWS - WIN)
        win = scores_ref[pl.ds(r0, WIN), :]
        pos = (r0 * 128
               + jax.lax.broadcasted_iota(jnp.int32, (WIN, 128), 0) * 128
               + jax.lax.broadcasted_iota(jnp.int32, (WIN, 128), 1))
        mv = jnp.where((pos >= s) & (pos < e), win, NEG)
        vals_ref[pl.ds(b, 1)] = mv[None]
        return carry

    jax.lax.fori_loop(0, B, build, 0)
    vt = jnp.swapaxes(vals_ref[...].reshape(GB, WIN * 128), 0, 1)

    posi = jax.lax.broadcasted_iota(jnp.int32, (WIN * 128, 1), 0)

    def step(k, carry):
        m = jnp.max(carry, axis=0, keepdims=True)
        cand = jnp.where(carry == m, posi, WIN * 128)
        wp = jnp.min(cand, axis=0, keepdims=True)
        picks_ref[pl.ds(k, 1), :] = r0v * 128 + wp
        return jnp.where(posi == wp, NEG, carry)

    jax.lax.fori_loop(0, K, step, vt)
    out_ref[...] = jnp.swapaxes(picks_ref[...], 0, 1)


def _topk(scores2, ends):
    return pl.pallas_call(
        _topk_body,
        grid=(1,),
        in_specs=[
            pl.BlockSpec((SROWS, 128), lambda i: (0, 0)),
            pl.BlockSpec(memory_space=pltpu.SMEM),
            pl.BlockSpec((8, GB), lambda i: (0, 0)),
        ],
        out_specs=pl.BlockSpec((GB, K), lambda i: (0, 0)),
        out_shape=jax.ShapeDtypeStruct((GB, K), jnp.int32),
        scratch_shapes=[pltpu.VMEM((GB, WIN, 128), jnp.float32),
                        pltpu.VMEM((K, GB), jnp.int32)],
    )(scores2, ends, ends)


NWORK = 25
RPW2 = (B * K) // NWORK


def _sc_gather(feat, idx):
    mesh = plsc.VectorSubcoreMesh(core_axis_name="c", subcore_axis_name="s")

    @functools.partial(
        pl.kernel,
        mesh=mesh,
        out_type=jax.ShapeDtypeStruct((B * K, D), jnp.float32),
        scratch_types=[
            pltpu.VMEM((RPW2,), jnp.int32),
            pltpu.VMEM((RPW2, D), jnp.float32),
            pltpu.SemaphoreType.DMA,
        ],
    )
    def k(feat_hbm, idx_hbm, out_hbm, idx_v, rows_v, sem):
        wid = jax.lax.axis_index("s") * 2 + jax.lax.axis_index("c")
        base = wid * RPW2

        @pl.when(wid < NWORK)
        def _():
            pltpu.sync_copy(idx_hbm.at[pl.ds(base, RPW2)], idx_v)
            pltpu.async_copy(feat_hbm.at[idx_v], rows_v, sem).wait()
            pltpu.sync_copy(rows_v, out_hbm.at[pl.ds(base, RPW2)])

    return k(feat, idx)


def _sort_body(x_ref, o_ref):
    x = x_ref[...]
    lane = jax.lax.broadcasted_iota(jnp.int32, (1, 128), 1)
    for kk in (2, 4, 8, 16, 32, 64, 128):
        j = kk // 2
        while j >= 1:
            bitj = (lane & j) != 0
            up = (lane & kk) == 0
            rolled_m = jnp.concatenate([x[:, j:], x[:, :j]], axis=1)
            rolled_p = jnp.concatenate([x[:, 128 - j:], x[:, : 128 - j]], axis=1)
            partner = jnp.where(bitj, rolled_p, rolled_m)
            take_min = jnp.logical_xor(bitj, up)
            x = jnp.where(take_min, jnp.minimum(x, partner),
                          jnp.maximum(x, partner))
            j //= 2
    o_ref[...] = x


def _sort_rows(rows):
    blk = 800
    return pl.pallas_call(
        _sort_body,
        grid=((B * K) // blk,),
        in_specs=[pl.BlockSpec((blk, 128), lambda i: (i, 0))],
        out_specs=pl.BlockSpec((blk, 128), lambda i: (i, 0)),
        out_shape=jax.ShapeDtypeStruct((B * K, 128), jnp.float32),
    )(rows)


def kernel(feat, graph_ids):
    gids = graph_ids.astype(jnp.int32)
    gpad = jnp.pad(gids, (0, NPAD - N), constant_values=B).reshape(GRID, 8, 128)
    scores2, ends = _rowmax(feat, gpad)
    picks_t = _topk(scores2, ends)
    rows = _sc_gather(feat, picks_t.reshape(GB * K))
    srt = _sort_rows(rows)
    return srt.reshape(B, K * D)


# speedup vs baseline: 19.6784x; 1.1407x over previous
import functools

import jax
import jax.numpy as jnp
from jax.experimental import pallas as pl
from jax.experimental.pallas import tpu as pltpu
from jax.experimental.pallas import tpu_sc as plsc

N = 100000
D = 128
B = 100
K = 64

BLK = 1024
GRID = (N + BLK - 1) // BLK
NPAD = GRID * BLK
SROWS = NPAD // 128
WIN = 16
GB = 128
NEG = float("-inf")

NW = 32
RPW = (B * K) // NW


def _rowmax_body(feat_ref, gid_ref, scores_ref, ends_ref, acc_ref):
    i = pl.program_id(0)
    x = feat_ref[...]
    scores_ref[...] = jnp.max(x.reshape(BLK // 128, 128, 128), axis=-1)

    g = gid_ref[...][0]
    bv = jax.lax.broadcasted_iota(jnp.int32, (GB, 1, 1), 0)
    inc = (g[None] <= bv).astype(jnp.int32)

    @pl.when(i == 0)
    def _():
        acc_ref[...] = inc

    @pl.when(i > 0)
    def _():
        acc_ref[...] = acc_ref[...] + inc

    @pl.when(i == GRID - 1)
    def _():
        cnt = jnp.sum(acc_ref[...], axis=(1, 2)).reshape(1, GB)
        ends_ref[...] = jnp.broadcast_to(cnt, (8, GB))


def _rowmax(feat, gpad):
    return pl.pallas_call(
        _rowmax_body,
        grid=(GRID,),
        in_specs=[
            pl.BlockSpec((BLK, 128), lambda i: (i, 0)),
            pl.BlockSpec((1, 8, 128), lambda i: (i, 0, 0)),
        ],
        out_specs=[
            pl.BlockSpec((8, 128), lambda i: (i, 0)),
            pl.BlockSpec((8, GB), lambda i: (0, 0)),
        ],
        out_shape=[
            jax.ShapeDtypeStruct((SROWS, 128), jnp.float32),
            jax.ShapeDtypeStruct((8, GB), jnp.int32),
        ],
        scratch_shapes=[pltpu.VMEM((GB, 8, 128), jnp.int32)],
    )(feat, gpad)


def _topk_body(scores_ref, ends_s_ref, ends_v_ref, out_ref, vals_ref, picks_ref):
    ev = ends_v_ref[0:1, :]
    sv = jnp.concatenate([jnp.zeros((1, 1), jnp.int32), ev[:, : GB - 1]], axis=1)
    r0v = jnp.clip(sv // 128, 0, SROWS - WIN)

    vals_ref[...] = jnp.full((GB, WIN, 128), NEG, jnp.float32)

    def build(b, carry):
        e = ends_s_ref[0, b]
        s = jnp.where(b == 0, 0, ends_s_ref[0, jnp.maximum(b - 1, 0)])
        r0 = jnp.clip(s // 128, 0, SROWS - WIN)
        win = scores_ref[pl.ds(r0, WIN), :]
        pos = (r0 * 128
               + jax.lax.broadcasted_iota(jnp.int32, (WIN, 128), 0) * 128
               + jax.lax.broadcasted_iota(jnp.int32, (WIN, 128), 1))
        mv = jnp.where((pos >= s) & (pos < e), win, NEG)
        vals_ref[pl.ds(b, 1)] = mv[None]
        return carry

    jax.lax.fori_loop(0, B, build, 0)
    vt = jnp.swapaxes(vals_ref[...].reshape(GB, WIN * 128), 0, 1)

    posi = jax.lax.broadcasted_iota(jnp.int32, (WIN * 128, 1), 0)

    def step(k, carry):
        m = jnp.max(carry, axis=0, keepdims=True)
        cand = jnp.where(carry == m, posi, WIN * 128)
        wp = jnp.min(cand, axis=0, keepdims=True)
        picks_ref[pl.ds(k, 1), :] = r0v * 128 + wp
        return jnp.where(posi == wp, NEG, carry)

    jax.lax.fori_loop(0, K, step, vt)
    out_ref[...] = jnp.swapaxes(picks_ref[...], 0, 1)


def _topk(scores2, ends):
    return pl.pallas_call(
        _topk_body,
        grid=(1,),
        in_specs=[
            pl.BlockSpec((SROWS, 128), lambda i: (0, 0)),
            pl.BlockSpec(memory_space=pltpu.SMEM),
            pl.BlockSpec((8, GB), lambda i: (0, 0)),
        ],
        out_specs=pl.BlockSpec((GB, K), lambda i: (0, 0)),
        out_shape=jax.ShapeDtypeStruct((GB, K), jnp.int32),
        scratch_shapes=[pltpu.VMEM((GB, WIN, 128), jnp.float32),
                        pltpu.VMEM((K, GB), jnp.int32)],
    )(scores2, ends, ends)


NWORK = 25
RPW2 = (B * K) // NWORK


def _sc_gather(feat, idx):
    mesh = plsc.VectorSubcoreMesh(core_axis_name="c", subcore_axis_name="s")

    @functools.partial(
        pl.kernel,
        mesh=mesh,
        out_type=jax.ShapeDtypeStruct((B * K, D), jnp.float32),
        scratch_types=[
            pltpu.VMEM((RPW2,), jnp.int32),
            pltpu.VMEM((RPW2, D), jnp.float32),
            pltpu.SemaphoreType.DMA,
        ],
    )
    def k(feat_hbm, idx_hbm, out_hbm, idx_v, rows_v, sem):
        wid = jax.lax.axis_index("s") * 2 + jax.lax.axis_index("c")
        base = wid * RPW2

        @pl.when(wid < NWORK)
        def _():
            pltpu.sync_copy(idx_hbm.at[pl.ds(base, RPW2)], idx_v)
            pltpu.async_copy(feat_hbm.at[idx_v], rows_v, sem).wait()
            pltpu.sync_copy(rows_v, out_hbm.at[pl.ds(base, RPW2)])

    return k(feat, idx)


def _sort_body(x_ref, o_ref):
    xt = jnp.swapaxes(x_ref[...], 0, 1)
    d = jax.lax.broadcasted_iota(jnp.int32, (128, 1), 0)
    for kk in (2, 4, 8, 16, 32, 64, 128):
        j = kk // 2
        while j >= 1:
            bitj = (d & j) != 0
            up = (d & kk) == 0
            rm = jnp.concatenate([xt[j:], xt[:j]], axis=0)
            rp = jnp.concatenate([xt[128 - j:], xt[: 128 - j]], axis=0)
            partner = jnp.where(bitj, rp, rm)
            take_min = jnp.logical_xor(bitj, up)
            xt = jnp.where(take_min, jnp.minimum(xt, partner),
                           jnp.maximum(xt, partner))
            j //= 2
    o_ref[...] = jnp.swapaxes(xt, 0, 1)


def _sort_rows(rows):
    blk = 640
    return pl.pallas_call(
        _sort_body,
        grid=((B * K) // blk,),
        in_specs=[pl.BlockSpec((blk, 128), lambda i: (i, 0))],
        out_specs=pl.BlockSpec((blk, 128), lambda i: (i, 0)),
        out_shape=jax.ShapeDtypeStruct((B * K, 128), jnp.float32),
    )(rows)


def kernel(feat, graph_ids):
    gids = graph_ids.astype(jnp.int32)
    gpad = jnp.pad(gids, (0, NPAD - N), constant_values=B).reshape(GRID, 8, 128)
    scores2, ends = _rowmax(feat, gpad)
    picks_t = _topk(scores2, ends)
    rows = _sc_gather(feat, picks_t.reshape(GB * K))
    srt = _sort_rows(rows)
    return srt.reshape(B, K * D)
